# baseline (speedup 1.0000x reference)
import functools

import jax
import jax.numpy as jnp
from jax import lax
from jax.experimental import pallas as pl
from jax.experimental.pallas import tpu as pltpu
from jax.experimental.pallas import tpu_sc as plsc

_N = 10000
_E = 320000
_D_IN = 128
_D_EDGE = 4
_Z = 64
_CH = [16, 32, 64, 128]
_GSZ = 64

_NW = 32
_EPW = _E // _NW
_C = 80
_NCH = _EPW // _C
_RPT = _N // 16


def _groups(co):
  return [min(_GSZ, co - g) for g in range(0, co, _GSZ)]


def _make_edge_pass(cg):
  fcg = 5 * cg
  mesh = plsc.VectorSubcoreMesh(core_axis_name="c", subcore_axis_name="s")

  @functools.partial(
      pl.kernel,
      out_type=jax.ShapeDtypeStruct((2, _N, cg), jnp.float32),
      mesh=mesh,
      compiler_params=pltpu.CompilerParams(use_tc_tiling_on_sc=False),
      scratch_types=[
          pltpu.VMEM((_NCH, _C), jnp.int32),
          pltpu.VMEM((_NCH, _C), jnp.int32),
          [pltpu.VMEM((_C, 16), jnp.float32)] * 2,
          [pltpu.VMEM((_C, fcg), jnp.float32)] * 2,
          [pltpu.VMEM((_C, cg), jnp.float32)] * 2,
          pltpu.VMEM_SHARED((_N, cg), jnp.float32),
          [pltpu.SemaphoreType.DMA] * 2,
          [pltpu.SemaphoreType.DMA] * 2,
      ],
  )
  def edge_pass(tcat, src3, dst3, ea4, zeros, out,
                src_v, dst_v, ea_b, rows_b, msg_b, agg_sh, gsem, ssem):
    cid = lax.axis_index("c")
    sid = lax.axis_index("s")
    wid = sid * 2 + cid

    pltpu.sync_copy(zeros.at[pl.ds(sid * _RPT, _RPT)],
                    agg_sh.at[pl.ds(sid * _RPT, _RPT)])
    pltpu.sync_copy(src3.at[wid], src_v)
    pltpu.sync_copy(dst3.at[wid], dst_v)
    plsc.subcore_barrier()

    def issue(j, b):
      pltpu.async_copy(ea4.at[wid, j], ea_b[b], gsem[b])
      pltpu.async_copy(tcat.at[src_v.at[j]], rows_b[b], gsem[b])

    issue(0, 0)
    issue(1, 1)

    def process(i, j, b):
      ea_v, rows_v, msg_v = ea_b[b], rows_b[b], msg_b[b]
      pltpu.make_async_copy(ea4.at[wid, j], ea_v, gsem[b]).wait()
      pltpu.make_async_copy(tcat.at[src_v.at[j]], rows_v, gsem[b]).wait()

      def drain_scatter():
        pltpu.make_async_copy(
            msg_v, agg_sh.at[dst_v.at[j - 2]], ssem[b]).wait()
      if isinstance(i, int):
        if i >= 1:
          drain_scatter()
      else:
        pl.when(i >= 1)(drain_scatter)

      @plsc.parallel_loop(0, _C, unroll=4)
      def edge_body(e):
        wv = ea_v[e, pl.ds(0, 16)]
        w0 = wv[0]
        w1 = wv[1]
        w2 = wv[2]
        w3 = wv[3]
        for ov in range(cg // 16):
          acc = rows_v[e, pl.ds(4 * cg + ov * 16, 16)]
          acc = acc + w0 * rows_v[e, pl.ds(0 * cg + ov * 16, 16)]
          acc = acc + w1 * rows_v[e, pl.ds(1 * cg + ov * 16, 16)]
          acc = acc + w2 * rows_v[e, pl.ds(2 * cg + ov * 16, 16)]
          acc = acc + w3 * rows_v[e, pl.ds(3 * cg + ov * 16, 16)]
          msg_v[e, pl.ds(ov * 16, 16)] = acc

      pltpu.async_copy(msg_v, agg_sh.at[dst_v.at[j]], ssem[b], add=True)

      if isinstance(j, int):
        if j + 2 < _NCH:
          issue(j + 2, b)
      else:
        pl.when(j + 2 < _NCH)(lambda: issue(j + 2, b))

    def pair_body(i, carry):
      process(i, 2 * i, 0)
      process(i, 2 * i + 1, 1)
      return carry

    lax.fori_loop(0, _NCH // 2, pair_body, 0)
    if _NCH % 2:
      process(_NCH // 2, _NCH - 1, 0)
    pltpu.make_async_copy(
        msg_b[(_NCH - 1) % 2], agg_sh.at[dst_v.at[_NCH - 1]],
        ssem[(_NCH - 1) % 2]).wait()
    pltpu.make_async_copy(
        msg_b[(_NCH - 2) % 2], agg_sh.at[dst_v.at[_NCH - 2]],
        ssem[(_NCH - 2) % 2]).wait()
    plsc.subcore_barrier()
    pltpu.sync_copy(agg_sh.at[pl.ds(sid * _RPT, _RPT)],
                    out.at[cid, pl.ds(sid * _RPT, _RPT)])

  return edge_pass


def _bn_gelu(s, g, b):
  mu = jnp.mean(s, axis=0, keepdims=True)
  var = jnp.mean((s - mu) * (s - mu), axis=0, keepdims=True)
  h = (s - mu) * lax.rsqrt(var + 1e-5) * g + b
  return jax.nn.gelu(h)


def _make_mid_body(nga, ng):
  def body(*refs):
    agg_rs = refs[:nga]
    r_r, g_r, b_r = refs[nga:nga + 3]
    wcat_rs = refs[nga + 3:nga + 3 + ng]
    rw_r, rb_r = refs[nga + 3 + ng], refs[nga + 4 + ng]
    tcat_os = refs[nga + 5 + ng:nga + 5 + 2 * ng]
    r_o = refs[nga + 5 + 2 * ng]
    s = jnp.concatenate([a[0] + a[1] for a in agg_rs], axis=-1) + r_r[...]
    h = _bn_gelu(s, g_r[...], b_r[...])
    for t_o, w_r in zip(tcat_os, wcat_rs):
      t_o[...] = jnp.dot(h, w_r[...], preferred_element_type=jnp.float32)
    r_o[...] = jnp.dot(h, rw_r[...], preferred_element_type=jnp.float32) + rb_r[...]
  return body


def _make_headbn_body(nga):
  def body(*refs):
    agg_rs = refs[:nga]
    r_r, g_r, b_r, w1_r, b1_r, w2_r, b2_r, mu_o, sig_o = refs[nga:]
    s = jnp.concatenate([a[0] + a[1] for a in agg_rs], axis=-1) + r_r[...]
    h = _bn_gelu(s, g_r[...], b_r[...])
    hidden = jax.nn.gelu(
        jnp.dot(h, w1_r[...], preferred_element_type=jnp.float32) + b1_r[...])
    z = jnp.dot(hidden, w2_r[...], preferred_element_type=jnp.float32) + b2_r[...] + h
    mu_o[...] = z[:, :_Z]
    sig_o[...] = jnp.exp(jnp.clip(z[:, _Z:], -30.0, 20.0))
  return body


def _make_tf_body(ng):
  def body(*refs):
    h_r = refs[0]
    wcat_rs = refs[1:1 + ng]
    rw_r, rb_r = refs[1 + ng], refs[2 + ng]
    tcat_os = refs[3 + ng:3 + 2 * ng]
    r_o = refs[3 + 2 * ng]
    h = h_r[...]
    for t_o, w_r in zip(tcat_os, wcat_rs):
      t_o[...] = jnp.dot(h, w_r[...], preferred_element_type=jnp.float32)
    r_o[...] = jnp.dot(h, rw_r[...], preferred_element_type=jnp.float32) + rb_r[...]
  return body


def _tc_call(body, out_shapes, *args):
  return pl.pallas_call(
      body,
      out_shape=[jax.ShapeDtypeStruct(s, jnp.float32) for s in out_shapes],
  )(*args)


def _wcats(p, ci, co):
  w5 = jnp.concatenate([p["edge_w"], p["edge_b"][None, :]], axis=0)
  w5 = w5.reshape(5, ci, co)
  out = []
  for g in range(0, co, _GSZ):
    cg = min(_GSZ, co - g)
    out.append(w5[:, :, g:g + cg].transpose(1, 0, 2).reshape(ci, 5 * cg))
  return out


def kernel(x, edge_attr, params, edge_index, batch):
  src3 = edge_index[0].astype(jnp.int32).reshape(_NW, _NCH, _C)
  dst3 = edge_index[1].astype(jnp.int32).reshape(_NW, _NCH, _C)
  ea_r = edge_attr.astype(jnp.bfloat16).astype(jnp.float32)
  ea4 = jnp.pad(ea_r, ((0, 0), (0, 16 - _D_EDGE))).reshape(
      _NW, _NCH, _C, 16)

  ins = [_D_IN] + _CH[:-1]
  convs = [params["conv%d" % i] for i in range(len(_CH))]
  wcats = [_wcats(p, ci, co) for p, ci, co in zip(convs, ins, _CH)]

  p0 = convs[0]
  g0 = _groups(_CH[0])
  outs = _tc_call(
      _make_tf_body(len(g0)),
      [(_N, 5 * cg) for cg in g0] + [(_N, _CH[0])],
      x, *wcats[0], p0["root_w"], p0["root_b"][None, :])
  tcats, r = outs[:-1], outs[-1]

  for i, co in enumerate(_CH):
    p = convs[i]
    gs = _groups(co)
    aggs = [
        _make_edge_pass(cg)(tc, src3, dst3, ea4,
                            jnp.zeros((_N, cg), jnp.float32))
        for tc, cg in zip(tcats, gs)
    ]
    if i + 1 < len(_CH):
      gn = _groups(_CH[i + 1])
      pn = convs[i + 1]
      outs = _tc_call(
          _make_mid_body(len(gs), len(gn)),
          [(_N, 5 * cg) for cg in gn] + [(_N, _CH[i + 1])],
          *aggs, r, p["bn_g"][None, :], p["bn_b"][None, :],
          *wcats[i + 1], pn["root_w"], pn["root_b"][None, :])
      tcats, r = outs[:-1], outs[-1]
    else:
      ph = params["head"]
      z_mu, z_sigma = _tc_call(
          _make_headbn_body(len(gs)), [(_N, _Z), (_N, _Z)],
          *aggs, r, p["bn_g"][None, :], p["bn_b"][None, :],
          ph["w1"], ph["b1"][None, :], ph["w2"], ph["b2"][None, :])
  return z_mu, z_sigma


# speedup vs baseline: 1.0299x; 1.0299x over previous
import functools

import jax
import jax.numpy as jnp
from jax import lax
from jax.experimental import pallas as pl
from jax.experimental.pallas import tpu as pltpu
from jax.experimental.pallas import tpu_sc as plsc

_N = 10000
_E = 320000
_D_IN = 128
_D_EDGE = 4
_Z = 64
_CH = [16, 32, 64, 128]
_GSZ = 64

_NW = 32
_EPW = _E // _NW
_C = 80
_NCH = _EPW // _C
_RPT = _N // 16


def _groups(co):
  return [min(_GSZ, co - g) for g in range(0, co, _GSZ)]


def _make_edge_pass(cg):
  fcg = 5 * cg
  mesh = plsc.VectorSubcoreMesh(core_axis_name="c", subcore_axis_name="s")

  @functools.partial(
      pl.kernel,
      out_type=jax.ShapeDtypeStruct((2, _N, cg), jnp.float32),
      mesh=mesh,
      compiler_params=pltpu.CompilerParams(use_tc_tiling_on_sc=False),
      scratch_types=[
          pltpu.VMEM((_NCH, _C), jnp.int32),
          pltpu.VMEM((_NCH, _C), jnp.int32),
          [pltpu.VMEM((_C, 16), jnp.float32)] * 2,
          [pltpu.VMEM((_C, fcg), jnp.float32)] * 2,
          [pltpu.VMEM((_C, cg), jnp.float32)] * 2,
          pltpu.VMEM_SHARED((_N, cg), jnp.float32),
          [pltpu.SemaphoreType.DMA] * 2,
          [pltpu.SemaphoreType.DMA] * 2,
      ],
  )
  def edge_pass(tcat, src3, dst3, ea4, zeros, out,
                src_v, dst_v, ea_b, rows_b, msg_b, agg_sh, gsem, ssem):
    cid = lax.axis_index("c")
    sid = lax.axis_index("s")
    wid = sid * 2 + cid

    pltpu.sync_copy(zeros.at[pl.ds(sid * _RPT, _RPT)],
                    agg_sh.at[pl.ds(sid * _RPT, _RPT)])
    pltpu.sync_copy(src3.at[wid], src_v)
    pltpu.sync_copy(dst3.at[wid], dst_v)
    plsc.subcore_barrier()

    def issue(j, b):
      pltpu.async_copy(ea4.at[wid, j], ea_b[b], gsem[b])
      pltpu.async_copy(tcat.at[src_v.at[j]], rows_b[b], gsem[b])

    issue(0, 0)
    issue(1, 1)

    def process(i, j, b):
      ea_v, rows_v, msg_v = ea_b[b], rows_b[b], msg_b[b]
      pltpu.make_async_copy(ea4.at[wid, j], ea_v, gsem[b]).wait()
      pltpu.make_async_copy(tcat.at[src_v.at[j]], rows_v, gsem[b]).wait()

      def drain_scatter():
        pltpu.make_async_copy(
            msg_v, agg_sh.at[dst_v.at[j - 2]], ssem[b]).wait()
      if isinstance(i, int):
        if i >= 1:
          drain_scatter()
      else:
        pl.when(i >= 1)(drain_scatter)

      @plsc.parallel_loop(0, _C, unroll=2)
      def edge_body(e):
        wv = ea_v[e, pl.ds(0, 16)]
        w0 = wv[0]
        w1 = wv[1]
        w2 = wv[2]
        w3 = wv[3]
        for ov in range(cg // 16):
          acc = rows_v[e, pl.ds(4 * cg + ov * 16, 16)]
          acc = acc + w0 * rows_v[e, pl.ds(0 * cg + ov * 16, 16)]
          acc = acc + w1 * rows_v[e, pl.ds(1 * cg + ov * 16, 16)]
          acc = acc + w2 * rows_v[e, pl.ds(2 * cg + ov * 16, 16)]
          acc = acc + w3 * rows_v[e, pl.ds(3 * cg + ov * 16, 16)]
          msg_v[e, pl.ds(ov * 16, 16)] = acc

      pltpu.async_copy(msg_v, agg_sh.at[dst_v.at[j]], ssem[b], add=True)

      if isinstance(j, int):
        if j + 2 < _NCH:
          issue(j + 2, b)
      else:
        pl.when(j + 2 < _NCH)(lambda: issue(j + 2, b))

    def pair_body(i, carry):
      process(i, 2 * i, 0)
      process(i, 2 * i + 1, 1)
      return carry

    lax.fori_loop(0, _NCH // 2, pair_body, 0)
    if _NCH % 2:
      process(_NCH // 2, _NCH - 1, 0)
    pltpu.make_async_copy(
        msg_b[(_NCH - 1) % 2], agg_sh.at[dst_v.at[_NCH - 1]],
        ssem[(_NCH - 1) % 2]).wait()
    pltpu.make_async_copy(
        msg_b[(_NCH - 2) % 2], agg_sh.at[dst_v.at[_NCH - 2]],
        ssem[(_NCH - 2) % 2]).wait()
    plsc.subcore_barrier()
    pltpu.sync_copy(agg_sh.at[pl.ds(sid * _RPT, _RPT)],
                    out.at[cid, pl.ds(sid * _RPT, _RPT)])

  return edge_pass


def _bn_gelu(s, g, b):
  mu = jnp.mean(s, axis=0, keepdims=True)
  var = jnp.mean((s - mu) * (s - mu), axis=0, keepdims=True)
  h = (s - mu) * lax.rsqrt(var + 1e-5) * g + b
  return jax.nn.gelu(h)


def _make_mid_body(nga, ng):
  def body(*refs):
    agg_rs = refs[:nga]
    r_r, g_r, b_r = refs[nga:nga + 3]
    wcat_rs = refs[nga + 3:nga + 3 + ng]
    rw_r, rb_r = refs[nga + 3 + ng], refs[nga + 4 + ng]
    tcat_os = refs[nga + 5 + ng:nga + 5 + 2 * ng]
    r_o = refs[nga + 5 + 2 * ng]
    s = jnp.concatenate([a[0] + a[1] for a in agg_rs], axis=-1) + r_r[...]
    h = _bn_gelu(s, g_r[...], b_r[...])
    for t_o, w_r in zip(tcat_os, wcat_rs):
      t_o[...] = jnp.dot(h, w_r[...], preferred_element_type=jnp.float32)
    r_o[...] = jnp.dot(h, rw_r[...], preferred_element_type=jnp.float32) + rb_r[...]
  return body


def _make_headbn_body(nga):
  def body(*refs):
    agg_rs = refs[:nga]
    r_r, g_r, b_r, w1_r, b1_r, w2_r, b2_r, mu_o, sig_o = refs[nga:]
    s = jnp.concatenate([a[0] + a[1] for a in agg_rs], axis=-1) + r_r[...]
    h = _bn_gelu(s, g_r[...], b_r[...])
    hidden = jax.nn.gelu(
        jnp.dot(h, w1_r[...], preferred_element_type=jnp.float32) + b1_r[...])
    z = jnp.dot(hidden, w2_r[...], preferred_element_type=jnp.float32) + b2_r[...] + h
    mu_o[...] = z[:, :_Z]
    sig_o[...] = jnp.exp(jnp.clip(z[:, _Z:], -30.0, 20.0))
  return body


def _make_tf_body(ng):
  def body(*refs):
    h_r = refs[0]
    wcat_rs = refs[1:1 + ng]
    rw_r, rb_r = refs[1 + ng], refs[2 + ng]
    tcat_os = refs[3 + ng:3 + 2 * ng]
    r_o = refs[3 + 2 * ng]
    h = h_r[...]
    for t_o, w_r in zip(tcat_os, wcat_rs):
      t_o[...] = jnp.dot(h, w_r[...], preferred_element_type=jnp.float32)
    r_o[...] = jnp.dot(h, rw_r[...], preferred_element_type=jnp.float32) + rb_r[...]
  return body


def _tc_call(body, out_shapes, *args):
  return pl.pallas_call(
      body,
      out_shape=[jax.ShapeDtypeStruct(s, jnp.float32) for s in out_shapes],
  )(*args)


def _wcats(p, ci, co):
  w5 = jnp.concatenate([p["edge_w"], p["edge_b"][None, :]], axis=0)
  w5 = w5.reshape(5, ci, co)
  out = []
  for g in range(0, co, _GSZ):
    cg = min(_GSZ, co - g)
    out.append(w5[:, :, g:g + cg].transpose(1, 0, 2).reshape(ci, 5 * cg))
  return out


def kernel(x, edge_attr, params, edge_index, batch):
  src3 = edge_index[0].astype(jnp.int32).reshape(_NW, _NCH, _C)
  dst3 = edge_index[1].astype(jnp.int32).reshape(_NW, _NCH, _C)
  ea_r = edge_attr.astype(jnp.bfloat16).astype(jnp.float32)
  ea4 = jnp.pad(ea_r, ((0, 0), (0, 16 - _D_EDGE))).reshape(
      _NW, _NCH, _C, 16)

  ins = [_D_IN] + _CH[:-1]
  convs = [params["conv%d" % i] for i in range(len(_CH))]
  wcats = [_wcats(p, ci, co) for p, ci, co in zip(convs, ins, _CH)]

  p0 = convs[0]
  g0 = _groups(_CH[0])
  outs = _tc_call(
      _make_tf_body(len(g0)),
      [(_N, 5 * cg) for cg in g0] + [(_N, _CH[0])],
      x, *wcats[0], p0["root_w"], p0["root_b"][None, :])
  tcats, r = outs[:-1], outs[-1]

  for i, co in enumerate(_CH):
    p = convs[i]
    gs = _groups(co)
    aggs = [
        _make_edge_pass(cg)(tc, src3, dst3, ea4,
                            jnp.zeros((_N, cg), jnp.float32))
        for tc, cg in zip(tcats, gs)
    ]
    if i + 1 < len(_CH):
      gn = _groups(_CH[i + 1])
      pn = convs[i + 1]
      outs = _tc_call(
          _make_mid_body(len(gs), len(gn)),
          [(_N, 5 * cg) for cg in gn] + [(_N, _CH[i + 1])],
          *aggs, r, p["bn_g"][None, :], p["bn_b"][None, :],
          *wcats[i + 1], pn["root_w"], pn["root_b"][None, :])
      tcats, r = outs[:-1], outs[-1]
    else:
      ph = params["head"]
      z_mu, z_sigma = _tc_call(
          _make_headbn_body(len(gs)), [(_N, _Z), (_N, _Z)],
          *aggs, r, p["bn_g"][None, :], p["bn_b"][None, :],
          ph["w1"], ph["b1"][None, :], ph["w2"], ph["b2"][None, :])
  return z_mu, z_sigma
